# Initial kernel scaffold; baseline (speedup 1.0000x reference)
import jax
import jax.numpy as jnp
from jax.experimental import pallas as pl


def kernel(l1_xyz, l1_points, l2_xyz, l2_points, l3_xyz, l3_points, l4_xyz, l4_points, W1, b1, W2, b2, W3, b3, W4, b4):
    raise NotImplementedError("write your pallas kernel here")


# speedup vs baseline: 39.0159x; 39.0159x over previous
import jax
import jax.numpy as jnp
from jax.experimental import pallas as pl


def _topk_weights(d2, k, bandwidth):
    rem = d2
    sel = jnp.zeros(d2.shape, dtype=jnp.bool_)
    for _ in range(k):
        m = jnp.min(rem, axis=0, keepdims=True)
        hit = rem <= m
        sel = jnp.logical_or(sel, hit)
        rem = jnp.where(hit, jnp.inf, rem)
    w = jnp.where(sel, jnp.exp(-d2 / (2.0 * bandwidth * bandwidth)), 0.0)
    return w / (jnp.sum(w, axis=0, keepdims=True) + 1e-8)


def _stage(sxyzT, dxyz, spoints, dpoints, Wa, ba, Wb, bb, k, bandwidth):
    s2 = jnp.sum(sxyzT * sxyzT, axis=1, keepdims=True)
    d2n = jnp.sum(dxyz * dxyz, axis=0, keepdims=True)
    ab = jnp.dot(sxyzT, dxyz, preferred_element_type=jnp.float32)
    d2 = jnp.maximum(s2 + d2n - 2.0 * ab, 0.0)
    w = _topk_weights(d2, k, bandwidth)
    interp = jnp.dot(spoints, w, preferred_element_type=jnp.float32)
    new = interp + dpoints
    h = jnp.maximum(jnp.dot(Wa, new, preferred_element_type=jnp.float32) + ba, 0.0)
    return jnp.maximum(jnp.dot(Wb, h, preferred_element_type=jnp.float32) + bb, 0.0)


def _fused_body(l4xT_ref, l3x_ref, l4p_ref, l3p_ref, l3xT_ref, l2x_ref, l2p_ref,
                W1_ref, b1_ref, W2_ref, b2_ref, W3_ref, b3_ref, W4_ref, b4_ref,
                out_ref):
    bw = 0.05
    l3_new = _stage(l4xT_ref[0], l3x_ref[0], l4p_ref[0], l3p_ref[0],
                    W1_ref[...], b1_ref[...], W2_ref[...], b2_ref[...],
                    16, 8 * bw)
    out_ref[0] = _stage(l3xT_ref[0], l2x_ref[0], l3_new, l2p_ref[0],
                        W3_ref[...], b3_ref[...], W4_ref[...], b4_ref[...],
                        16, 4 * bw)


def kernel(l1_xyz, l1_points, l2_xyz, l2_points, l3_xyz, l3_points, l4_xyz,
           l4_points, W1, b1, W2, b2, W3, b3, W4, b4):
    del l1_xyz, l1_points
    B = l2_xyz.shape[0]
    l4xT = jnp.transpose(l4_xyz, (0, 2, 1))
    l3xT = jnp.transpose(l3_xyz, (0, 2, 1))
    b1c, b2c = b1[:, None], b2[:, None]
    b3c, b4c = b3[:, None], b4[:, None]

    def bspec(shape):
        return pl.BlockSpec((1,) + shape, lambda b: (b, 0, 0))

    def wspec(shape):
        return pl.BlockSpec(shape, lambda b: (0,) * len(shape))

    return pl.pallas_call(
        _fused_body,
        grid=(B,),
        in_specs=[
            bspec((64, 3)), bspec((3, 256)), bspec((512, 64)), bspec((512, 256)),
            bspec((256, 3)), bspec((3, 1024)), bspec((512, 1024)),
            wspec((512, 512)), wspec((512, 1)), wspec((512, 512)), wspec((512, 1)),
            wspec((256, 512)), wspec((256, 1)), wspec((256, 256)), wspec((256, 1)),
        ],
        out_specs=pl.BlockSpec((1, 256, 1024), lambda b: (b, 0, 0)),
        out_shape=jax.ShapeDtypeStruct((B, 256, 1024), jnp.float32),
    )(l4xT, l3_xyz, l4_points, l3_points, l3xT, l2_xyz, l2_points,
      W1, b1c, W2, b2c, W3, b3c, W4, b4c)
